# baseline (device time: 29947 ns/iter reference)
import jax
import jax.numpy as jnp
from jax import lax
from jax.experimental import pallas as pl
from jax.experimental.pallas import tpu as pltpu

N_DEV = 8
BLK = 256
S = 4
PIECE = BLK // S


def _gelu(y):
    c = 0.7978845608028654
    return 0.5 * y * (1.0 + jnp.tanh(c * (y + 0.044715 * y * y * y)))


def _decode(i):
    z = i // 4
    p = i % 4
    y = (p >= 2).astype(jnp.int32)
    x = ((p == 1) | (p == 2)).astype(jnp.int32)
    return x, y, z


def _encode(x, y, z):
    return z * 4 + (y * 3 + x - 2 * x * y)


def _neighbor(i, dx, dy, dz):
    x, y, z = _decode(i)
    return _encode(x ^ dx, y ^ dy, z ^ dz)


_MASKS = (
    (1, 0, 0),
    (0, 1, 0),
    (0, 0, 1),
    (1, 1, 0),
    (1, 0, 1),
    (0, 1, 1),
    (1, 1, 1),
)
_WAIT_ORDER = (0, 1, 2, 3, 5, 4, 6)


def kernel(x, w_mat):
    k, m_per = x.shape
    kw, n = w_mat.shape

    def body(x_ref, w_hbm, out_ref, w_ref, xloc_ref, gather_ref,
             send_sems, recv_sems, w_sem, xloc_sem):
        my = lax.axis_index("i")

        wcopy = pltpu.make_async_copy(w_hbm, w_ref, w_sem)
        wcopy.start()
        xcopy = pltpu.make_async_copy(
            x_ref.at[pl.ds(my * BLK, BLK), :], xloc_ref, xloc_sem
        )
        xcopy.start()

        bar = pltpu.get_barrier_semaphore()
        for d in range(1, N_DEV):
            peer = lax.rem(my + d, N_DEV)
            pl.semaphore_signal(
                bar, inc=1, device_id=(peer,),
                device_id_type=pl.DeviceIdType.MESH,
            )
        pl.semaphore_wait(bar, N_DEV - 1)

        rdmas = {}
        for p in range(S):
            for j in range(N_DEV - 1):
                dx, dy, dz = _MASKS[j]
                dst = _neighbor(my, dx, dy, dz)
                r = pltpu.make_async_remote_copy(
                    src_ref=x_ref.at[pl.ds(dst * BLK + p * PIECE, PIECE), :],
                    dst_ref=gather_ref.at[j, pl.ds(p * PIECE, PIECE), :],
                    send_sem=send_sems.at[p, j],
                    recv_sem=recv_sems.at[p, j],
                    device_id=(dst,),
                    device_id_type=pl.DeviceIdType.MESH,
                )
                r.start()
                rdmas[(p, j)] = r

        wcopy.wait()
        xcopy.wait()
        wloc = w_ref[pl.ds(my * BLK, BLK), :]
        out_ref[:, :] = jnp.dot(
            xloc_ref[:, :], wloc, preferred_element_type=jnp.float32
        )

        for p in range(S):
            for j in _WAIT_ORDER:
                rdmas[(p, j)].wait_recv()
                dx, dy, dz = _MASKS[j]
                src = _neighbor(my, dx, dy, dz)
                wblk = w_ref[pl.ds(src * BLK, BLK), :]
                out_ref[pl.ds(p * PIECE, PIECE), :] += jnp.dot(
                    gather_ref[j, pl.ds(p * PIECE, PIECE), :],
                    wblk,
                    preferred_element_type=jnp.float32,
                )

        out_ref[:, :] = _gelu(out_ref[:, :])

        for r in rdmas.values():
            r.wait_send()

    return pl.pallas_call(
        body,
        out_shape=jax.ShapeDtypeStruct((m_per, n), jnp.float32),
        in_specs=[
            pl.BlockSpec(memory_space=pltpu.MemorySpace.HBM),
            pl.BlockSpec(memory_space=pltpu.MemorySpace.HBM),
        ],
        out_specs=pl.BlockSpec(memory_space=pltpu.VMEM),
        scratch_shapes=[
            pltpu.VMEM((kw, n), jnp.float32),
            pltpu.VMEM((BLK, m_per), jnp.float32),
            pltpu.VMEM((N_DEV - 1, BLK, m_per), jnp.float32),
            pltpu.SemaphoreType.DMA((S, N_DEV - 1)),
            pltpu.SemaphoreType.DMA((S, N_DEV - 1)),
            pltpu.SemaphoreType.DMA,
            pltpu.SemaphoreType.DMA,
        ],
        compiler_params=pltpu.CompilerParams(collective_id=0),
    )(x, w_mat)


# device time: 25057 ns/iter; 1.1952x vs baseline; 1.1952x over previous
import jax
import jax.numpy as jnp
from jax import lax
from jax.experimental import pallas as pl
from jax.experimental.pallas import tpu as pltpu

N_DEV = 8
BLK = 256
S = 4
PIECE = BLK // S
_DUMMY_ROWS = 10752


def _gelu(y):
    c = 0.7978845608028654
    return 0.5 * y * (1.0 + jnp.tanh(c * (y + 0.044715 * y * y * y)))


def _decode(i):
    z = i // 4
    p = i % 4
    y = (p >= 2).astype(jnp.int32)
    x = ((p == 1) | (p == 2)).astype(jnp.int32)
    return x, y, z


def _encode(x, y, z):
    return z * 4 + (y * 3 + x - 2 * x * y)


def _neighbor(i, dx, dy, dz):
    x, y, z = _decode(i)
    return _encode(x ^ dx, y ^ dy, z ^ dz)


_MASKS = (
    (1, 0, 0),
    (0, 1, 0),
    (0, 0, 1),
    (1, 1, 0),
    (1, 0, 1),
    (0, 1, 1),
    (1, 1, 1),
)
_WAIT_ORDER = (0, 1, 2, 3, 5, 4, 6)


def kernel(x, w_mat):
    k, m_per = x.shape
    kw, n = w_mat.shape

    def body(x_ref, w_hbm, out_ref, w_ref, xloc_ref, gather_ref, dummy_ref,
             send_sems, recv_sems, w_sem, xloc_sem):
        my = lax.axis_index("i")

        wcopy = pltpu.make_async_copy(w_hbm, w_ref, w_sem)
        wcopy.start()
        xcopy = pltpu.make_async_copy(
            x_ref.at[pl.ds(my * BLK, BLK), :], xloc_ref, xloc_sem
        )
        xcopy.start()

        bar = pltpu.get_barrier_semaphore()
        for d in range(1, N_DEV):
            peer = lax.rem(my + d, N_DEV)
            pl.semaphore_signal(
                bar, inc=1, device_id=(peer,),
                device_id_type=pl.DeviceIdType.MESH,
            )
        pl.semaphore_wait(bar, N_DEV - 1)

        rdmas = {}
        for p in range(S):
            for j in range(N_DEV - 1):
                dx, dy, dz = _MASKS[j]
                dst = _neighbor(my, dx, dy, dz)
                r = pltpu.make_async_remote_copy(
                    src_ref=x_ref.at[pl.ds(dst * BLK + p * PIECE, PIECE), :],
                    dst_ref=gather_ref.at[j, pl.ds(p * PIECE, PIECE), :],
                    send_sem=send_sems.at[p, j],
                    recv_sem=recv_sems.at[p, j],
                    device_id=(dst,),
                    device_id_type=pl.DeviceIdType.MESH,
                )
                r.start()
                rdmas[(p, j)] = r

        wcopy.wait()
        xcopy.wait()
        wloc = w_ref[pl.ds(my * BLK, BLK), :]
        out_ref[:, :] = jnp.dot(
            xloc_ref[:, :], wloc, preferred_element_type=jnp.float32
        )

        for p in range(S):
            for j in _WAIT_ORDER:
                rdmas[(p, j)].wait_recv()
                dx, dy, dz = _MASKS[j]
                src = _neighbor(my, dx, dy, dz)
                wblk = w_ref[pl.ds(src * BLK, BLK), :]
                out_ref[pl.ds(p * PIECE, PIECE), :] += jnp.dot(
                    gather_ref[j, pl.ds(p * PIECE, PIECE), :],
                    wblk,
                    preferred_element_type=jnp.float32,
                )

        out_ref[:, :] = _gelu(out_ref[:, :])

        for r in rdmas.values():
            r.wait_send()

    return pl.pallas_call(
        body,
        out_shape=jax.ShapeDtypeStruct((m_per, n), jnp.float32),
        in_specs=[
            pl.BlockSpec(memory_space=pltpu.MemorySpace.HBM),
            pl.BlockSpec(memory_space=pltpu.MemorySpace.HBM),
        ],
        out_specs=pl.BlockSpec(memory_space=pltpu.VMEM),
        scratch_shapes=[
            pltpu.VMEM((kw, n), jnp.float32),
            pltpu.VMEM((BLK, m_per), jnp.float32),
            pltpu.VMEM((N_DEV - 1, BLK, m_per), jnp.float32),
            pltpu.VMEM((_DUMMY_ROWS, 1024), jnp.float32),
            pltpu.SemaphoreType.DMA((S, N_DEV - 1)),
            pltpu.SemaphoreType.DMA((S, N_DEV - 1)),
            pltpu.SemaphoreType.DMA,
            pltpu.SemaphoreType.DMA,
        ],
        compiler_params=pltpu.CompilerParams(
            collective_id=0,
            vmem_limit_bytes=67108864,
        ),
    )(x, w_mat)


# device time: 23534 ns/iter; 1.2725x vs baseline; 1.0647x over previous
import jax
import jax.numpy as jnp
from jax import lax
from jax.experimental import pallas as pl
from jax.experimental.pallas import tpu as pltpu

N_DEV = 8
BLK = 256
S = 4
PIECE = BLK // S
_DUMMY_ROWS = 10752


def _gelu(y):
    c = 0.7978845608028654
    return 0.5 * y * (1.0 + jnp.tanh(c * (y + 0.044715 * y * y * y)))


def _decode(i):
    z = i // 4
    p = i % 4
    y = (p >= 2).astype(jnp.int32)
    x = ((p == 1) | (p == 2)).astype(jnp.int32)
    return x, y, z


def _encode(x, y, z):
    return z * 4 + (y * 3 + x - 2 * x * y)


def _neighbor(i, dx, dy, dz):
    x, y, z = _decode(i)
    return _encode(x ^ dx, y ^ dy, z ^ dz)


_MASKS = (
    (1, 0, 0),
    (0, 1, 0),
    (0, 0, 1),
    (1, 1, 0),
    (1, 0, 1),
    (0, 1, 1),
    (1, 1, 1),
)
_WAIT_ORDER = (0, 1, 2, 3, 5, 4, 6)


def kernel(x, w_mat):
    k, m_per = x.shape
    kw, n = w_mat.shape

    def body(x_ref, w_hbm, out_hbm, acc_ref, w_ref, xloc_ref, gather_ref,
             dummy_ref, send_sems, recv_sems, out_sems, w_sem, xloc_sem):
        my = lax.axis_index("i")

        wcopy = pltpu.make_async_copy(w_hbm, w_ref, w_sem)
        wcopy.start()
        xcopy = pltpu.make_async_copy(
            x_ref.at[pl.ds(my * BLK, BLK), :], xloc_ref, xloc_sem
        )
        xcopy.start()

        bar = pltpu.get_barrier_semaphore()
        for d in range(1, N_DEV):
            peer = lax.rem(my + d, N_DEV)
            pl.semaphore_signal(
                bar, inc=1, device_id=(peer,),
                device_id_type=pl.DeviceIdType.MESH,
            )
        pl.semaphore_wait(bar, N_DEV - 1)

        rdmas = {}
        for p in range(S):
            for j in range(N_DEV - 1):
                dx, dy, dz = _MASKS[j]
                dst = _neighbor(my, dx, dy, dz)
                r = pltpu.make_async_remote_copy(
                    src_ref=x_ref.at[pl.ds(dst * BLK + p * PIECE, PIECE), :],
                    dst_ref=gather_ref.at[j, pl.ds(p * PIECE, PIECE), :],
                    send_sem=send_sems.at[p, j],
                    recv_sem=recv_sems.at[p, j],
                    device_id=(dst,),
                    device_id_type=pl.DeviceIdType.MESH,
                )
                r.start()
                rdmas[(p, j)] = r

        wcopy.wait()
        xcopy.wait()
        wloc = w_ref[pl.ds(my * BLK, BLK), :]
        acc_ref[:, :] = jnp.dot(
            xloc_ref[:, :], wloc, preferred_element_type=jnp.float32
        )

        outcopies = []
        for p in range(S):
            rows = pl.ds(p * PIECE, PIECE)
            for j in _WAIT_ORDER:
                rdmas[(p, j)].wait_recv()
                dx, dy, dz = _MASKS[j]
                src = _neighbor(my, dx, dy, dz)
                wblk = w_ref[pl.ds(src * BLK, BLK), :]
                acc_ref[rows, :] += jnp.dot(
                    gather_ref[j, rows, :],
                    wblk,
                    preferred_element_type=jnp.float32,
                )
            acc_ref[rows, :] = _gelu(acc_ref[rows, :])
            ocp = pltpu.make_async_copy(
                acc_ref.at[rows, :], out_hbm.at[rows, :], out_sems.at[p]
            )
            ocp.start()
            outcopies.append(ocp)

        for ocp in outcopies:
            ocp.wait()
        for r in rdmas.values():
            r.wait_send()

    return pl.pallas_call(
        body,
        out_shape=jax.ShapeDtypeStruct((m_per, n), jnp.float32),
        in_specs=[
            pl.BlockSpec(memory_space=pltpu.MemorySpace.HBM),
            pl.BlockSpec(memory_space=pltpu.MemorySpace.HBM),
        ],
        out_specs=pl.BlockSpec(memory_space=pltpu.MemorySpace.HBM),
        scratch_shapes=[
            pltpu.VMEM((m_per, n), jnp.float32),
            pltpu.VMEM((kw, n), jnp.float32),
            pltpu.VMEM((BLK, m_per), jnp.float32),
            pltpu.VMEM((N_DEV - 1, BLK, m_per), jnp.float32),
            pltpu.VMEM((_DUMMY_ROWS, 1024), jnp.float32),
            pltpu.SemaphoreType.DMA((S, N_DEV - 1)),
            pltpu.SemaphoreType.DMA((S, N_DEV - 1)),
            pltpu.SemaphoreType.DMA((S,)),
            pltpu.SemaphoreType.DMA,
            pltpu.SemaphoreType.DMA,
        ],
        compiler_params=pltpu.CompilerParams(
            collective_id=0,
            vmem_limit_bytes=67108864,
        ),
    )(x, w_mat)
